# baseline (speedup 1.0000x reference)
import jax
import jax.numpy as jnp
from jax import lax
from jax.experimental import pallas as pl
from jax.experimental.pallas import tpu as pltpu
from jax.experimental.pallas import tpu_sc as plsc

DIM = 512
CODE_SIZE = 16384
CODE_DEPTH = 8
MAX_LEN = 2048
B = 16
VOCAB = CODE_SIZE + 2
SEQ = MAX_LEN
HALF = SEQ // 2
CHUNK = 64
NCHUNK = HALF // CHUNK
NWAVE = 2
WCHUNK = NCHUNK // NWAVE
LANES = 16
VPR = DIM // LANES


def _sc_body(toks, emb, eot1, eot2, lens,
             out1, out2, mask,
             idx_v, x0, x1, acc_v, mask_v, len_v,
             sx0, sx1, so, sa):
    b = lax.axis_index("s")
    h = lax.axis_index("c")
    r0w = h * HALF

    pltpu.sync_copy(lens.at[b], len_v)
    iota = lax.iota(jnp.int32, LANES)
    lv = len_v[...]
    l2 = jnp.minimum(lv + 1, MAX_LEN)

    def _mrow(v, _):
        pos = iota + (r0w + v * LANES)
        mask_v[pl.ds(v * LANES, LANES)] = jnp.where(pos < l2, 1, 0)
        return 0
    lax.fori_loop(0, HALF // LANES, _mrow, 0)
    pltpu.sync_copy(mask_v, mask.at[b, pl.ds(r0w, HALF)])

    def _copy_pass(dst, src):
        def _row(r, _):
            for c in range(VPR):
                sl = pl.ds(c * LANES, LANES)
                dst[r, sl] = src[r, sl]
            return 0
        lax.fori_loop(0, CHUNK, _row, 0)

    def _add_pass(dst, src):
        def _row(r, _):
            for c in range(VPR):
                sl = pl.ds(c * LANES, LANES)
                dst[r, sl] = dst[r, sl] + src[r, sl]
            return 0
        lax.fori_loop(0, CHUNK, _row, 0)

    def _gather(k, jl, buf, sem):
        return pltpu.async_copy(emb.at[idx_v.at[k, jl]], buf, sem)

    def _chunk(jg, jl, guard):
        r0 = r0w + jg * CHUNK

        _gather(0, jl, x0, sx0)
        g1 = _gather(1, jl, x1, sx1)
        pltpu.make_async_copy(emb.at[idx_v.at[0, jl]], x0, sx0).wait()
        pltpu.async_copy(x0, out1.at[b, pl.ds(r0, CHUNK)], so)
        g1.wait()

        def _reclaim():
            pltpu.make_async_copy(acc_v, out2.at[b, pl.ds(r0, CHUNK)], sa).wait()
        if guard is None:
            _reclaim()
        else:
            pl.when(guard)(_reclaim)

        _copy_pass(acc_v, x1)
        _gather(2, jl, x1, sx1)
        pltpu.make_async_copy(emb.at[idx_v.at[2, jl]], x1, sx1).wait()
        _add_pass(acc_v, x1)
        _gather(4, jl, x1, sx1)
        pltpu.make_async_copy(x0, out1.at[b, pl.ds(r0, CHUNK)], so).wait()
        _gather(3, jl, x0, sx0)
        ring = [(x1, sx1), (x0, sx0)]
        for k in range(3, CODE_DEPTH):
            pb, ps = ring[k % 2]
            pltpu.make_async_copy(emb.at[idx_v.at[k, jl]], pb, ps).wait()
            _add_pass(acc_v, pb)
            if k + 2 < CODE_DEPTH:
                _gather(k + 2, jl, pb, ps)
        pltpu.async_copy(acc_v, out2.at[b, pl.ds(r0, CHUNK)], sa)

    for w in range(NWAVE):
        pltpu.sync_copy(
            toks.at[b, :, pl.ds(h * NCHUNK + w * WCHUNK, WCHUNK), :], idx_v)

        def _off(j, _):
            for k in range(1, CODE_DEPTH):
                for c in range(CHUNK // LANES):
                    sl = pl.ds(c * LANES, LANES)
                    idx_v[k, j, sl] = idx_v[k, j, sl] + k * VOCAB
            return 0
        lax.fori_loop(0, WCHUNK, _off, 0)

        if w == 0:
            def _loop(j, _):
                _chunk(j, j, j >= 1)
                return 0
        else:
            def _loop(j, _):
                _chunk(w * WCHUNK + j, j, None)
                return 0
        lax.fori_loop(0, WCHUNK, _loop, 0)

    pltpu.make_async_copy(acc_v, out2.at[b, pl.ds(r0w, CHUNK)], sa).wait()

    @pl.when(h == 0)
    def _():
        pltpu.sync_copy(eot1, x0.at[pl.ds(0, 1)])
        pltpu.sync_copy(x0.at[pl.ds(0, 1)], out1.at[b, pl.ds(0, 1)])
        pltpu.sync_copy(eot2, x0.at[pl.ds(0, 1)])
        pltpu.sync_copy(x0.at[pl.ds(0, 1)], out2.at[b, pl.ds(0, 1)])


@jax.jit
def _run(toks, emb_flat, eot1, eot2, lens):
    kfn = pl.kernel(
        _sc_body,
        out_type=(
            jax.ShapeDtypeStruct((B, SEQ, DIM), jnp.float32),
            jax.ShapeDtypeStruct((B, SEQ, DIM), jnp.float32),
            jax.ShapeDtypeStruct((B, SEQ), jnp.int32),
        ),
        mesh=plsc.VectorSubcoreMesh(core_axis_name="c", subcore_axis_name="s"),
        scratch_types=[
            pltpu.VMEM((CODE_DEPTH, WCHUNK, CHUNK), jnp.int32),
            pltpu.VMEM((CHUNK, DIM), jnp.float32),
            pltpu.VMEM((CHUNK, DIM), jnp.float32),
            pltpu.VMEM((CHUNK, DIM), jnp.float32),
            pltpu.VMEM((HALF,), jnp.int32),
            pltpu.VMEM((LANES,), jnp.int32),
        ] + [pltpu.SemaphoreType.DMA] * 4,
    )
    return kfn(toks, emb_flat, eot1, eot2, lens)


def kernel(tokens, lengths, emb, EOT_emb, layer2_EOT_emb):
    tok = tokens.reshape(B, CODE_DEPTH, MAX_LEN - 1)
    pad = jnp.full((B, CODE_DEPTH, 1), CODE_SIZE + 1, jnp.int32)
    toks = jnp.concatenate([pad, tok], axis=2)
    toks = toks.reshape(B, CODE_DEPTH, NCHUNK * 2, CHUNK)
    emb_flat = emb.reshape(CODE_DEPTH * VOCAB, DIM)
    lrep = jnp.broadcast_to(lengths[:, None], (B, LANES))
    return _run(toks, emb_flat, EOT_emb, layer2_EOT_emb, lrep)


# speedup vs baseline: 1.0743x; 1.0743x over previous
import jax
import jax.numpy as jnp
from jax import lax
from jax.experimental import pallas as pl
from jax.experimental.pallas import tpu as pltpu
from jax.experimental.pallas import tpu_sc as plsc

DIM = 512
CODE_SIZE = 16384
CODE_DEPTH = 8
MAX_LEN = 2048
B = 16
VOCAB = CODE_SIZE + 2
SEQ = MAX_LEN
HALF = SEQ // 2
CHUNK = 64
NCHUNK = HALF // CHUNK
NWAVE = 2
WCHUNK = NCHUNK // NWAVE
LANES = 16
VPR = DIM // LANES


def _sc_body(toks, emb, eot1, eot2, lens,
             out1, out2, mask,
             idx_v, x0, x1, acc_v, mask_v, len_v,
             sx0, sx1, so, sa, sga):
    b = lax.axis_index("s")
    h = lax.axis_index("c")
    r0w = h * HALF

    pltpu.sync_copy(lens.at[b], len_v)
    iota = lax.iota(jnp.int32, LANES)
    lv = len_v[...]
    l2 = jnp.minimum(lv + 1, MAX_LEN)

    def _mrow(v, _):
        pos = iota + (r0w + v * LANES)
        mask_v[pl.ds(v * LANES, LANES)] = jnp.where(pos < l2, 1, 0)
        return 0
    lax.fori_loop(0, HALF // LANES, _mrow, 0)
    pltpu.sync_copy(mask_v, mask.at[b, pl.ds(r0w, HALF)])

    def _copy_pass(dst, src):
        def _row(r, _):
            for c in range(VPR):
                sl = pl.ds(c * LANES, LANES)
                dst[r, sl] = src[r, sl]
            return 0
        lax.fori_loop(0, CHUNK, _row, 0)

    def _add_pass(dst, src):
        def _row(r, _):
            for c in range(VPR):
                sl = pl.ds(c * LANES, LANES)
                dst[r, sl] = dst[r, sl] + src[r, sl]
            return 0
        lax.fori_loop(0, CHUNK, _row, 0)

    def _gather(k, jl, buf, sem):
        return pltpu.async_copy(emb.at[idx_v.at[k, jl]], buf, sem)

    def _chunk(jg, jl, guard):
        r0 = r0w + jg * CHUNK

        _gather(0, jl, x0, sx0)
        g1 = _gather(1, jl, x1, sx1)
        pltpu.make_async_copy(emb.at[idx_v.at[0, jl]], x0, sx0).wait()
        pltpu.async_copy(x0, out1.at[b, pl.ds(r0, CHUNK)], so)

        def _reclaim():
            pltpu.make_async_copy(acc_v, out2.at[b, pl.ds(r0, CHUNK)], sa).wait()
        if guard is None:
            _reclaim()
        else:
            pl.when(guard)(_reclaim)
        _gather(2, jl, acc_v, sga)

        g1.wait()
        pltpu.make_async_copy(x0, out1.at[b, pl.ds(r0, CHUNK)], so).wait()
        _gather(3, jl, x0, sx0)
        pltpu.make_async_copy(emb.at[idx_v.at[2, jl]], acc_v, sga).wait()
        _add_pass(acc_v, x1)
        _gather(4, jl, x1, sx1)
        ring = [(x1, sx1), (x0, sx0)]
        for k in range(3, CODE_DEPTH):
            pb, ps = ring[k % 2]
            pltpu.make_async_copy(emb.at[idx_v.at[k, jl]], pb, ps).wait()
            _add_pass(acc_v, pb)
            if k + 2 < CODE_DEPTH:
                _gather(k + 2, jl, pb, ps)
        pltpu.async_copy(acc_v, out2.at[b, pl.ds(r0, CHUNK)], sa)

    for w in range(NWAVE):
        pltpu.sync_copy(
            toks.at[b, :, pl.ds(h * NCHUNK + w * WCHUNK, WCHUNK), :], idx_v)

        def _off(j, _):
            for k in range(1, CODE_DEPTH):
                for c in range(CHUNK // LANES):
                    sl = pl.ds(c * LANES, LANES)
                    idx_v[k, j, sl] = idx_v[k, j, sl] + k * VOCAB
            return 0
        lax.fori_loop(0, WCHUNK, _off, 0)

        if w == 0:
            def _loop(j, _):
                _chunk(j, j, j >= 1)
                return 0
        else:
            def _loop(j, _):
                _chunk(w * WCHUNK + j, j, None)
                return 0
        lax.fori_loop(0, WCHUNK, _loop, 0)

    pltpu.make_async_copy(acc_v, out2.at[b, pl.ds(r0w, CHUNK)], sa).wait()

    @pl.when(h == 0)
    def _():
        pltpu.sync_copy(eot1, x0.at[pl.ds(0, 1)])
        pltpu.sync_copy(x0.at[pl.ds(0, 1)], out1.at[b, pl.ds(0, 1)])
        pltpu.sync_copy(eot2, x0.at[pl.ds(0, 1)])
        pltpu.sync_copy(x0.at[pl.ds(0, 1)], out2.at[b, pl.ds(0, 1)])


@jax.jit
def _run(toks, emb_flat, eot1, eot2, lens):
    kfn = pl.kernel(
        _sc_body,
        out_type=(
            jax.ShapeDtypeStruct((B, SEQ, DIM), jnp.float32),
            jax.ShapeDtypeStruct((B, SEQ, DIM), jnp.float32),
            jax.ShapeDtypeStruct((B, SEQ), jnp.int32),
        ),
        mesh=plsc.VectorSubcoreMesh(core_axis_name="c", subcore_axis_name="s"),
        scratch_types=[
            pltpu.VMEM((CODE_DEPTH, WCHUNK, CHUNK), jnp.int32),
            pltpu.VMEM((CHUNK, DIM), jnp.float32),
            pltpu.VMEM((CHUNK, DIM), jnp.float32),
            pltpu.VMEM((CHUNK, DIM), jnp.float32),
            pltpu.VMEM((HALF,), jnp.int32),
            pltpu.VMEM((LANES,), jnp.int32),
        ] + [pltpu.SemaphoreType.DMA] * 5,
    )
    return kfn(toks, emb_flat, eot1, eot2, lens)


def kernel(tokens, lengths, emb, EOT_emb, layer2_EOT_emb):
    tok = tokens.reshape(B, CODE_DEPTH, MAX_LEN - 1)
    pad = jnp.full((B, CODE_DEPTH, 1), CODE_SIZE + 1, jnp.int32)
    toks = jnp.concatenate([pad, tok], axis=2)
    toks = toks.reshape(B, CODE_DEPTH, NCHUNK * 2, CHUNK)
    emb_flat = emb.reshape(CODE_DEPTH * VOCAB, DIM)
    lrep = jnp.broadcast_to(lengths[:, None], (B, LANES))
    return _run(toks, emb_flat, EOT_emb, layer2_EOT_emb, lrep)


# speedup vs baseline: 1.0768x; 1.0023x over previous
import jax
import jax.numpy as jnp
from jax import lax
from jax.experimental import pallas as pl
from jax.experimental.pallas import tpu as pltpu
from jax.experimental.pallas import tpu_sc as plsc

DIM = 512
CODE_SIZE = 16384
CODE_DEPTH = 8
MAX_LEN = 2048
B = 16
VOCAB = CODE_SIZE + 2
SEQ = MAX_LEN
HALF = SEQ // 2
CHUNK = 64
NCHUNK = HALF // CHUNK
NWAVE = 2
WCHUNK = NCHUNK // NWAVE
LANES = 16
VPR = DIM // LANES


def _sc_body(toks, emb, eot1, eot2, lens,
             out1, out2, mask,
             idx_v, x0, x1, acc_v, mask_v, len_v,
             sx0, sx1, so, sa, sga):
    b = lax.axis_index("s")
    h = lax.axis_index("c")
    r0w = h * HALF

    pltpu.sync_copy(lens.at[b], len_v)
    iota = lax.iota(jnp.int32, LANES)
    lv = len_v[...]
    l2 = jnp.minimum(lv + 1, MAX_LEN)

    def _mrow(v, _):
        pos = iota + (r0w + v * LANES)
        mask_v[pl.ds(v * LANES, LANES)] = jnp.where(pos < l2, 1, 0)
        return 0
    lax.fori_loop(0, HALF // LANES, _mrow, 0)
    pltpu.sync_copy(mask_v, mask.at[b, pl.ds(r0w, HALF)])

    def _copy_pass(dst, src):
        def _row(r, _):
            for c in range(VPR):
                sl = pl.ds(c * LANES, LANES)
                dst[r, sl] = src[r, sl]
            return 0
        lax.fori_loop(0, CHUNK, _row, 0)

    def _add_pass(dst, src):
        def _row(r, _):
            for c in range(VPR):
                sl = pl.ds(c * LANES, LANES)
                plsc.addupdate(dst.at[r, sl], src[r, sl])
            return 0
        lax.fori_loop(0, CHUNK, _row, 0)

    def _gather(k, jl, buf, sem):
        return pltpu.async_copy(emb.at[idx_v.at[k, jl]], buf, sem)

    def _chunk(jg, jl, guard):
        r0 = r0w + jg * CHUNK

        _gather(0, jl, x0, sx0)
        g1 = _gather(1, jl, x1, sx1)
        pltpu.make_async_copy(emb.at[idx_v.at[0, jl]], x0, sx0).wait()
        pltpu.async_copy(x0, out1.at[b, pl.ds(r0, CHUNK)], so)

        def _reclaim():
            pltpu.make_async_copy(acc_v, out2.at[b, pl.ds(r0, CHUNK)], sa).wait()
        if guard is None:
            _reclaim()
        else:
            pl.when(guard)(_reclaim)
        _gather(2, jl, acc_v, sga)

        g1.wait()
        pltpu.make_async_copy(x0, out1.at[b, pl.ds(r0, CHUNK)], so).wait()
        _gather(3, jl, x0, sx0)
        pltpu.make_async_copy(emb.at[idx_v.at[2, jl]], acc_v, sga).wait()
        _add_pass(acc_v, x1)
        _gather(4, jl, x1, sx1)
        ring = [(x1, sx1), (x0, sx0)]
        for k in range(3, CODE_DEPTH):
            pb, ps = ring[k % 2]
            pltpu.make_async_copy(emb.at[idx_v.at[k, jl]], pb, ps).wait()
            _add_pass(acc_v, pb)
            if k + 2 < CODE_DEPTH:
                _gather(k + 2, jl, pb, ps)
        pltpu.async_copy(acc_v, out2.at[b, pl.ds(r0, CHUNK)], sa)

    for w in range(NWAVE):
        pltpu.sync_copy(
            toks.at[b, :, pl.ds(h * NCHUNK + w * WCHUNK, WCHUNK), :], idx_v)

        def _off(j, _):
            for k in range(1, CODE_DEPTH):
                for c in range(CHUNK // LANES):
                    sl = pl.ds(c * LANES, LANES)
                    idx_v[k, j, sl] = idx_v[k, j, sl] + k * VOCAB
            return 0
        lax.fori_loop(0, WCHUNK, _off, 0)

        if w == 0:
            def _loop(j, _):
                _chunk(j, j, j >= 1)
                return 0
        else:
            def _loop(j, _):
                _chunk(w * WCHUNK + j, j, None)
                return 0
        lax.fori_loop(0, WCHUNK, _loop, 0)

    pltpu.make_async_copy(acc_v, out2.at[b, pl.ds(r0w, CHUNK)], sa).wait()

    @pl.when(h == 0)
    def _():
        pltpu.sync_copy(eot1, x0.at[pl.ds(0, 1)])
        pltpu.sync_copy(x0.at[pl.ds(0, 1)], out1.at[b, pl.ds(0, 1)])
        pltpu.sync_copy(eot2, x0.at[pl.ds(0, 1)])
        pltpu.sync_copy(x0.at[pl.ds(0, 1)], out2.at[b, pl.ds(0, 1)])


@jax.jit
def _run(toks, emb_flat, eot1, eot2, lens):
    kfn = pl.kernel(
        _sc_body,
        out_type=(
            jax.ShapeDtypeStruct((B, SEQ, DIM), jnp.float32),
            jax.ShapeDtypeStruct((B, SEQ, DIM), jnp.float32),
            jax.ShapeDtypeStruct((B, SEQ), jnp.int32),
        ),
        mesh=plsc.VectorSubcoreMesh(core_axis_name="c", subcore_axis_name="s"),
        scratch_types=[
            pltpu.VMEM((CODE_DEPTH, WCHUNK, CHUNK), jnp.int32),
            pltpu.VMEM((CHUNK, DIM), jnp.float32),
            pltpu.VMEM((CHUNK, DIM), jnp.float32),
            pltpu.VMEM((CHUNK, DIM), jnp.float32),
            pltpu.VMEM((HALF,), jnp.int32),
            pltpu.VMEM((LANES,), jnp.int32),
        ] + [pltpu.SemaphoreType.DMA] * 5,
    )
    return kfn(toks, emb_flat, eot1, eot2, lens)


def kernel(tokens, lengths, emb, EOT_emb, layer2_EOT_emb):
    tok = tokens.reshape(B, CODE_DEPTH, MAX_LEN - 1)
    pad = jnp.full((B, CODE_DEPTH, 1), CODE_SIZE + 1, jnp.int32)
    toks = jnp.concatenate([pad, tok], axis=2)
    toks = toks.reshape(B, CODE_DEPTH, NCHUNK * 2, CHUNK)
    emb_flat = emb.reshape(CODE_DEPTH * VOCAB, DIM)
    lrep = jnp.broadcast_to(lengths[:, None], (B, LANES))
    return _run(toks, emb_flat, EOT_emb, layer2_EOT_emb, lrep)
